# Initial kernel scaffold; baseline (speedup 1.0000x reference)
import jax
import jax.numpy as jnp
from jax.experimental import pallas as pl


def kernel(inputs, table):
    raise NotImplementedError("write your pallas kernel here")


# speedup vs baseline: 1.4779x; 1.4779x over previous
import functools

import jax
import jax.numpy as jnp
from jax import lax
from jax.experimental import pallas as pl
from jax.experimental.pallas import tpu as pltpu
from jax.experimental.pallas import tpu_sc as plsc

EMBED = 32

IDX_W = 128
GATHERS_PER_GROUP = 8
GROUP_ROWS = IDX_W * GATHERS_PER_GROUP


def _make_lookup(n_idx: int):
    info = plsc.get_sparse_core_info()
    nw = info.num_cores * info.num_subcores
    per_w = n_idx // nw
    assert per_w % GROUP_ROWS == 0
    n_groups = per_w // GROUP_ROWS
    idx_rows = per_w // IDX_W

    mesh = plsc.VectorSubcoreMesh(core_axis_name="c", subcore_axis_name="s")

    @functools.partial(
        pl.kernel,
        out_type=jax.ShapeDtypeStruct((n_idx, EMBED), jnp.float32),
        mesh=mesh,
        scratch_types=[
            pltpu.VMEM((idx_rows, IDX_W), jnp.int32),
            pltpu.VMEM((GROUP_ROWS, EMBED), jnp.float32),
            pltpu.SemaphoreType.DMA,
        ],
        compiler_params=pltpu.CompilerParams(use_tc_tiling_on_sc=False),
    )
    def lookup(idx_hbm, table_hbm, out_hbm, idx_v, buf, gsem):
        wid = lax.axis_index("s") * info.num_cores + lax.axis_index("c")
        pltpu.sync_copy(idx_hbm.at[pl.ds(wid * idx_rows, idx_rows)], idx_v)
        out_base = wid * per_w

        @pl.loop(0, n_groups)
        def _group(g):
            cps = []
            for j in range(GATHERS_PER_GROUP):
                row = g * GATHERS_PER_GROUP + j
                cps.append(
                    pltpu.async_copy(
                        table_hbm.at[idx_v.at[row]],
                        buf.at[pl.ds(j * IDX_W, IDX_W)],
                        gsem,
                    )
                )
            for cp in cps:
                cp.wait()
            pltpu.sync_copy(
                buf,
                out_hbm.at[pl.ds(out_base + g * GROUP_ROWS, GROUP_ROWS)],
            )

    return lookup


def kernel(inputs, table):
    b, l = inputs.shape
    n_idx = b * l
    idx = inputs.reshape(n_idx // IDX_W, IDX_W).astype(jnp.int32)
    out = _make_lookup(n_idx)(idx, table)
    return out.reshape(b, l, EMBED)


# speedup vs baseline: 1.4943x; 1.0111x over previous
import functools

import jax
import jax.numpy as jnp
from jax import lax
from jax.experimental import pallas as pl
from jax.experimental.pallas import tpu as pltpu
from jax.experimental.pallas import tpu_sc as plsc

EMBED = 32

IDX_W = 128
GATHERS_PER_GROUP = 8
GROUP_ROWS = IDX_W * GATHERS_PER_GROUP


def _make_lookup(n_idx: int):
    info = plsc.get_sparse_core_info()
    nw = info.num_cores * info.num_subcores
    per_w = n_idx // nw
    assert per_w % GROUP_ROWS == 0
    n_groups = per_w // GROUP_ROWS
    idx_rows = per_w // IDX_W

    mesh = plsc.VectorSubcoreMesh(core_axis_name="c", subcore_axis_name="s")

    @functools.partial(
        pl.kernel,
        out_type=jax.ShapeDtypeStruct((n_idx, EMBED), jnp.float32),
        mesh=mesh,
        scratch_types=[
            pltpu.VMEM((idx_rows, IDX_W), jnp.int32),
            pltpu.VMEM((2 * GROUP_ROWS, EMBED), jnp.float32),
            pltpu.SemaphoreType.DMA,
            pltpu.SemaphoreType.DMA,
        ],
        compiler_params=pltpu.CompilerParams(use_tc_tiling_on_sc=False),
    )
    def lookup(idx_hbm, table_hbm, out_hbm, idx_v, buf, gsem, ssem):
        wid = lax.axis_index("s") * info.num_cores + lax.axis_index("c")
        pltpu.sync_copy(idx_hbm.at[pl.ds(wid * idx_rows, idx_rows)], idx_v)
        out_base = wid * per_w

        def half(g):
            return buf.at[pl.ds((g % 2) * GROUP_ROWS, GROUP_ROWS)]

        def fire_gathers(g):
            base = (g % 2) * GROUP_ROWS
            for j in range(GATHERS_PER_GROUP):
                pltpu.async_copy(
                    table_hbm.at[idx_v.at[g * GATHERS_PER_GROUP + j]],
                    buf.at[pl.ds(base + j * IDX_W, IDX_W)],
                    gsem,
                )

        def drain(sem, g):
            pltpu.make_async_copy(
                out_hbm.at[pl.ds(0, GROUP_ROWS)], half(g), sem
            ).wait()

        fire_gathers(0)

        @pl.loop(0, n_groups)
        def _group(g):
            drain(gsem, g)

            @pl.when(g > 0)
            def _():
                drain(ssem, g - 1)

            @pl.when(g < n_groups - 1)
            def _():
                fire_gathers(g + 1)

            pltpu.async_copy(
                half(g),
                out_hbm.at[pl.ds(out_base + g * GROUP_ROWS, GROUP_ROWS)],
                ssem,
            )

        drain(ssem, n_groups - 1)

    return lookup


def kernel(inputs, table):
    b, l = inputs.shape
    n_idx = b * l
    idx = inputs.reshape(n_idx // IDX_W, IDX_W).astype(jnp.int32)
    out = _make_lookup(n_idx)(idx, table)
    return out.reshape(b, l, EMBED)
